# baseline (speedup 1.0000x reference)
import functools

import jax
import jax.numpy as jnp
import numpy as np
from jax import lax
from jax.experimental import pallas as pl
from jax.experimental.pallas import tpu as pltpu
from jax.experimental.pallas import tpu_sc as plsc

_CPAD = 128


def _diag_index_np(h, w):
    idx = []
    for s in range(h + w - 1):
        for i in range(h):
            j = s - i
            if 0 <= j < w:
                idx.append(i * w + j)
    return np.asarray(idx, dtype=np.int32)


def _transpose_body(x_ref, eye_ref, o_ref):
    c, h_blk, w = x_ref.shape[1:]
    xt = lax.dot_general(
        x_ref[0].reshape(c, h_blk * w), eye_ref[...],
        dimension_numbers=(((0,), (0,)), ((), ())),
        preferred_element_type=jnp.float32,
    )
    o_ref[...] = xt


def _untranspose_body(g_ref, eye_ref, o_ref):
    o_ref[0] = lax.dot_general(
        eye_ref[...], g_ref[...],
        dimension_numbers=(((1,), (1,)), ((), ())),
        preferred_element_type=jnp.float32,
    )


def _untranspose_alias_body(g_ref, eye_ref, acc_ref, o_ref):
    del acc_ref
    o_ref[0] = lax.dot_general(
        eye_ref[...], g_ref[...],
        dimension_numbers=(((1,), (1,)), ((), ())),
        preferred_element_type=jnp.float32,
    )


_NBUF = 2


@functools.lru_cache(maxsize=None)
def _make_sc_gather(n_rows, n_per_w, chunk):
    mesh = plsc.VectorSubcoreMesh(core_axis_name="c", subcore_axis_name="s")
    num_cores = mesh.num_cores
    n_chunks = n_per_w // chunk

    @functools.partial(
        pl.kernel,
        out_type=jax.ShapeDtypeStruct((n_rows, _CPAD), jnp.float32),
        mesh=mesh,
        scratch_types=[
            pltpu.VMEM((n_per_w,), jnp.int32),
            [pltpu.VMEM((chunk, _CPAD), jnp.float32) for _ in range(_NBUF)],
            [pltpu.SemaphoreType.DMA for _ in range(_NBUF)],
            [pltpu.SemaphoreType.DMA for _ in range(_NBUF)],
        ],
    )
    def gather_kernel(table_hbm, idx_hbm, out_hbm, idx_v, rows, gsems, ssems):
        wid = lax.axis_index("s") * num_cores + lax.axis_index("c")
        base = wid * n_per_w
        pltpu.sync_copy(idx_hbm.at[pl.ds(base, n_per_w)], idx_v)

        @pl.loop(0, n_chunks, step=_NBUF)
        def _chunks(ci):
            gets = []
            for k in range(_NBUF):
                idx_slice = idx_v.at[pl.ds((ci + k) * chunk, chunk)]
                gets.append(
                    pltpu.async_copy(table_hbm.at[idx_slice], rows[k], gsems[k]))
            puts = []
            for k in range(_NBUF):
                gets[k].wait()
                dst = out_hbm.at[pl.ds(base + (ci + k) * chunk, chunk)]
                puts.append(pltpu.async_copy(rows[k], dst, ssems[k]))
            for k in range(_NBUF):
                puts[k].wait()

    return gather_kernel


def kernel(x):
    b, c, h, w = x.shape
    hw = h * w
    idx_np = _diag_index_np(h, w)
    index = jnp.asarray(idx_np)
    gidx = jnp.asarray(idx_np)

    eye = jnp.asarray(np.eye(c, _CPAD, dtype=np.float32))
    info = plsc.get_sparse_core_info()
    n_workers = info.num_cores * info.num_subcores
    gather = _make_sc_gather(hw, hw // n_workers, 256)

    t_blk = 8192
    c_blk = 8192
    out = None
    for bi in range(b):
        h_blk = t_blk // w
        table = pl.pallas_call(
            _transpose_body,
            grid=(h // h_blk,),
            in_specs=[
                pl.BlockSpec((1, c, h_blk, w), lambda k, _bi=bi: (_bi, 0, k, 0)),
                pl.BlockSpec((c, _CPAD), lambda k: (0, 0)),
            ],
            out_specs=pl.BlockSpec((t_blk, _CPAD), lambda k: (k, 0)),
            out_shape=jax.ShapeDtypeStruct((hw, _CPAD), x.dtype),
        )(x, eye)
        g = gather(table, gidx)
        if out is None:
            out = pl.pallas_call(
                _untranspose_body,
                grid=(hw // c_blk,),
                in_specs=[
                    pl.BlockSpec((c_blk, _CPAD), lambda k: (k, 0)),
                    pl.BlockSpec((c, _CPAD), lambda k: (0, 0)),
                ],
                out_specs=pl.BlockSpec((1, c, c_blk), lambda k, _bi=bi: (_bi, 0, k)),
                out_shape=jax.ShapeDtypeStruct((b, c, hw), x.dtype),
            )(g, eye)
        else:
            out = pl.pallas_call(
                _untranspose_alias_body,
                grid=(hw // c_blk,),
                in_specs=[
                    pl.BlockSpec((c_blk, _CPAD), lambda k: (k, 0)),
                    pl.BlockSpec((c, _CPAD), lambda k: (0, 0)),
                    pl.BlockSpec(memory_space=pl.ANY),
                ],
                out_specs=pl.BlockSpec((1, c, c_blk), lambda k, _bi=bi: (_bi, 0, k)),
                out_shape=jax.ShapeDtypeStruct((b, c, hw), x.dtype),
                input_output_aliases={2: 0},
            )(g, eye, out)

    return jnp.transpose(out, (0, 2, 1)), index


# speedup vs baseline: 1.0199x; 1.0199x over previous
import functools

import jax
import jax.numpy as jnp
import numpy as np
from jax import lax
from jax.experimental import pallas as pl
from jax.experimental.pallas import tpu as pltpu
from jax.experimental.pallas import tpu_sc as plsc

_CPAD = 128


def _diag_index_np(h, w):
    idx = []
    for s in range(h + w - 1):
        for i in range(h):
            j = s - i
            if 0 <= j < w:
                idx.append(i * w + j)
    return np.asarray(idx, dtype=np.int32)


def _transpose_body(x_ref, eye_ref, o_ref):
    c, h_blk, w = x_ref.shape[1:]
    xt = lax.dot_general(
        x_ref[0].reshape(c, h_blk * w), eye_ref[...],
        dimension_numbers=(((0,), (0,)), ((), ())),
        preferred_element_type=jnp.float32,
    )
    o_ref[...] = xt


def _untranspose_body(g_ref, eye_ref, o_ref):
    o_ref[0] = lax.dot_general(
        eye_ref[...], g_ref[...],
        dimension_numbers=(((1,), (1,)), ((), ())),
        preferred_element_type=jnp.float32,
    )


def _untranspose_alias_body(g_ref, eye_ref, acc_ref, o_ref):
    del acc_ref
    o_ref[0] = lax.dot_general(
        eye_ref[...], g_ref[...],
        dimension_numbers=(((1,), (1,)), ((), ())),
        preferred_element_type=jnp.float32,
    )


_NBUF = 4


@functools.lru_cache(maxsize=None)
def _make_sc_gather(n_rows, n_per_w, chunk):
    mesh = plsc.VectorSubcoreMesh(core_axis_name="c", subcore_axis_name="s")
    num_cores = mesh.num_cores
    n_chunks = n_per_w // chunk

    @functools.partial(
        pl.kernel,
        out_type=jax.ShapeDtypeStruct((n_rows, _CPAD), jnp.float32),
        mesh=mesh,
        scratch_types=[
            pltpu.VMEM((n_per_w,), jnp.int32),
            [pltpu.VMEM((chunk, _CPAD), jnp.float32) for _ in range(_NBUF)],
            [pltpu.SemaphoreType.DMA for _ in range(_NBUF)],
            [pltpu.SemaphoreType.DMA for _ in range(_NBUF)],
        ],
    )
    def gather_kernel(table_hbm, idx_hbm, out_hbm, idx_v, rows, gsems, ssems):
        wid = lax.axis_index("s") * num_cores + lax.axis_index("c")
        base = wid * n_per_w
        pltpu.sync_copy(idx_hbm.at[pl.ds(base, n_per_w)], idx_v)

        @pl.loop(0, n_chunks, step=_NBUF)
        def _chunks(ci):
            gets = []
            for k in range(_NBUF):
                idx_slice = idx_v.at[pl.ds((ci + k) * chunk, chunk)]
                gets.append(
                    pltpu.async_copy(table_hbm.at[idx_slice], rows[k], gsems[k]))
            puts = []
            for k in range(_NBUF):
                gets[k].wait()
                dst = out_hbm.at[pl.ds(base + (ci + k) * chunk, chunk)]
                puts.append(pltpu.async_copy(rows[k], dst, ssems[k]))
            for k in range(_NBUF):
                puts[k].wait()

    return gather_kernel


def kernel(x):
    b, c, h, w = x.shape
    hw = h * w
    idx_np = _diag_index_np(h, w)
    index = jnp.asarray(idx_np)
    gidx = jnp.asarray(idx_np)

    eye = jnp.asarray(np.eye(c, _CPAD, dtype=np.float32))
    info = plsc.get_sparse_core_info()
    n_workers = info.num_cores * info.num_subcores
    gather = _make_sc_gather(hw, hw // n_workers, 128)

    t_blk = 16384
    c_blk = 16384
    out = None
    for bi in range(b):
        h_blk = t_blk // w
        table = pl.pallas_call(
            _transpose_body,
            grid=(h // h_blk,),
            in_specs=[
                pl.BlockSpec((1, c, h_blk, w), lambda k, _bi=bi: (_bi, 0, k, 0)),
                pl.BlockSpec((c, _CPAD), lambda k: (0, 0)),
            ],
            out_specs=pl.BlockSpec((t_blk, _CPAD), lambda k: (k, 0)),
            out_shape=jax.ShapeDtypeStruct((hw, _CPAD), x.dtype),
        )(x, eye)
        g = gather(table, gidx)
        if out is None:
            out = pl.pallas_call(
                _untranspose_body,
                grid=(hw // c_blk,),
                in_specs=[
                    pl.BlockSpec((c_blk, _CPAD), lambda k: (k, 0)),
                    pl.BlockSpec((c, _CPAD), lambda k: (0, 0)),
                ],
                out_specs=pl.BlockSpec((1, c, c_blk), lambda k, _bi=bi: (_bi, 0, k)),
                out_shape=jax.ShapeDtypeStruct((b, c, hw), x.dtype),
            )(g, eye)
        else:
            out = pl.pallas_call(
                _untranspose_alias_body,
                grid=(hw // c_blk,),
                in_specs=[
                    pl.BlockSpec((c_blk, _CPAD), lambda k: (k, 0)),
                    pl.BlockSpec((c, _CPAD), lambda k: (0, 0)),
                    pl.BlockSpec(memory_space=pl.ANY),
                ],
                out_specs=pl.BlockSpec((1, c, c_blk), lambda k, _bi=bi: (_bi, 0, k)),
                out_shape=jax.ShapeDtypeStruct((b, c, hw), x.dtype),
                input_output_aliases={2: 0},
            )(g, eye, out)

    return jnp.transpose(out, (0, 2, 1)), index
